# baseline (speedup 1.0000x reference)
import functools

import jax
import jax.numpy as jnp
import numpy as np
from jax import lax
from jax.experimental import pallas as pl
from jax.experimental.pallas import tpu as pltpu
from jax.experimental.pallas import tpu_sc as plsc

B, N, D = 8, 2048, 512


def _compute_order() -> np.ndarray:
    with jax.default_device(jax.local_devices(backend="cpu")[0]):
        rand = jax.random.uniform(jax.random.key(42), (B, N), dtype=jnp.float32)
        order = jnp.argsort(rand, axis=1)
        return np.asarray(order)


_ORDER = _compute_order()
_FLAT_IDX = (_ORDER.astype(np.int64) + np.arange(B, dtype=np.int64)[:, None] * N)
_FLAT_IDX = _FLAT_IDX.astype(np.int32).reshape(-1)

_INFO = plsc.get_sparse_core_info()
_NC, _NS, _L = _INFO.num_cores, _INFO.num_subcores, _INFO.num_lanes
_NW = _NC * _NS
_RPW = (B * N) // _NW
_CHUNK = 64
_NCHUNK = _RPW // _CHUNK
_NBUF = 3
_LAG = 1
_WPB = _NW // B

_ORDER_FLAT = _ORDER.astype(np.int32).reshape(-1)

_mesh = plsc.VectorSubcoreMesh(core_axis_name="c", subcore_axis_name="s")


@functools.partial(
    pl.kernel,
    mesh=_mesh,
    out_type=(
        jax.ShapeDtypeStruct((B, N, D), jnp.float32),
        jax.ShapeDtypeStruct((B, N), jnp.int32),
        jax.ShapeDtypeStruct((B, N), jnp.int32),
    ),
    scratch_types=[
        pltpu.VMEM((_RPW,), jnp.int32),
        pltpu.VMEM((_NBUF, _CHUNK, D), jnp.float32),
        pltpu.VMEM((_RPW,), jnp.int32),
        pltpu.VMEM((_RPW,), jnp.int32),
        pltpu.SemaphoreType.DMA,
        pltpu.SemaphoreType.DMA,
        pltpu.SemaphoreType.DMA,
        pltpu.SemaphoreType.DMA,
    ],
)
def _shuffle_sc(pf_hbm, tf_hbm, idx_hbm, out_p, out_t, out_o,
                idx_v, rows_v, tout_v, ord_v, gsem, ssem, tsem, osem):
    c = lax.axis_index("c")
    s = lax.axis_index("s")
    wid = s * _NC + c
    base = wid * _RPW
    b = wid // _WPB
    r0 = (wid % _WPB) * _RPW

    pltpu.sync_copy(idx_hbm.at[pl.ds(base, _RPW)], idx_v)

    def gather(j):
        return pltpu.async_copy(pf_hbm.at[idx_v.at[pl.ds(j * _CHUNK, _CHUNK)]],
                                rows_v.at[j % _NBUF], gsem)

    def scatter(j):
        return pltpu.async_copy(rows_v.at[j % _NBUF],
                                out_p.at[b, pl.ds(r0 + j * _CHUNK, _CHUNK)],
                                ssem)

    gcp, scp = {}, {}
    for j in range(_NCHUNK):
        if j - _NBUF >= 0:
            scp[j - _NBUF].wait()
        gcp[j] = gather(j)
        if j - _LAG >= 0:
            gcp[j - _LAG].wait()
            scp[j - _LAG] = scatter(j - _LAG)
    for j in range(_NCHUNK - _LAG, _NCHUNK):
        gcp[j].wait()
        scp[j] = scatter(j)

    tcopies = [
        pltpu.async_copy(tf_hbm.at[idx_v.at[pl.ds(j * _CHUNK, _CHUNK)]],
                         tout_v.at[pl.ds(j * _CHUNK, _CHUNK)], tsem)
        for j in range(_NCHUNK)
    ]

    boff = b * N
    for t in range(_RPW // _L):
        ord_v[pl.ds(t * _L, _L)] = idx_v[pl.ds(t * _L, _L)] - boff
    ocp = pltpu.async_copy(ord_v, out_o.at[b, pl.ds(r0, _RPW)], osem)

    for j in range(max(0, _NCHUNK - _NBUF), _NCHUNK):
        scp[j].wait()
    for cp in tcopies:
        cp.wait()
    pltpu.sync_copy(tout_v, out_t.at[b, pl.ds(r0, _RPW)])
    ocp.wait()


def kernel(patches, tokens):
    pf = patches.reshape(B * N, D)
    tf = tokens.reshape(B * N)
    idx = jnp.asarray(_FLAT_IDX)
    out_p, out_t, out_o = _shuffle_sc(pf, tf, idx)
    return (out_p, out_t, out_o)


# speedup vs baseline: 1.0056x; 1.0056x over previous
import functools

import jax
import jax.numpy as jnp
import numpy as np
from jax import lax
from jax.experimental import pallas as pl
from jax.experimental.pallas import tpu as pltpu
from jax.experimental.pallas import tpu_sc as plsc

B, N, D = 8, 2048, 512


def _compute_order() -> np.ndarray:
    with jax.default_device(jax.local_devices(backend="cpu")[0]):
        rand = jax.random.uniform(jax.random.key(42), (B, N), dtype=jnp.float32)
        order = jnp.argsort(rand, axis=1)
        return np.asarray(order)


_ORDER = _compute_order()
_FLAT_IDX = (_ORDER.astype(np.int64) + np.arange(B, dtype=np.int64)[:, None] * N)
_FLAT_IDX = _FLAT_IDX.astype(np.int32).reshape(-1)

_INFO = plsc.get_sparse_core_info()
_NC, _NS, _L = _INFO.num_cores, _INFO.num_subcores, _INFO.num_lanes
_NW = _NC * _NS
_RPW = (B * N) // _NW
_CHUNK = 64
_NCHUNK = _RPW // _CHUNK
_NBUF = 3
_LAG = 1
_WPB = _NW // B

_ORDER_FLAT = _ORDER.astype(np.int32).reshape(-1)

_mesh = plsc.VectorSubcoreMesh(core_axis_name="c", subcore_axis_name="s")


@functools.partial(
    pl.kernel,
    mesh=_mesh,
    out_type=(
        jax.ShapeDtypeStruct((B, N, D), jnp.float32),
        jax.ShapeDtypeStruct((B, N), jnp.int32),
        jax.ShapeDtypeStruct((B, N), jnp.int32),
    ),
    scratch_types=[
        pltpu.VMEM((_RPW,), jnp.int32),
        pltpu.VMEM((_NBUF, _CHUNK, D), jnp.float32),
        pltpu.VMEM((_RPW,), jnp.int32),
        pltpu.VMEM((_RPW,), jnp.int32),
        pltpu.SemaphoreType.DMA,
        pltpu.SemaphoreType.DMA,
        pltpu.SemaphoreType.DMA,
        pltpu.SemaphoreType.DMA,
    ],
)
def _shuffle_sc(pf_hbm, tf_hbm, idx_hbm, out_p, out_t, out_o,
                idx_v, rows_v, tout_v, ord_v, gsem, ssem, tsem, osem):
    c = lax.axis_index("c")
    s = lax.axis_index("s")
    wid = s * _NC + c
    base = wid * _RPW
    b = wid // _WPB
    r0 = (wid % _WPB) * _RPW

    pltpu.sync_copy(idx_hbm.at[pl.ds(base, _RPW)], idx_v)

    def gather(j):
        return pltpu.async_copy(pf_hbm.at[idx_v.at[pl.ds(j * _CHUNK, _CHUNK)]],
                                rows_v.at[j % _NBUF], gsem)

    def scatter(j):
        return pltpu.async_copy(rows_v.at[j % _NBUF],
                                out_p.at[b, pl.ds(r0 + j * _CHUNK, _CHUNK)],
                                ssem)

    gcp, scp = {}, {}
    for j in range(_NCHUNK):
        if j - _NBUF >= 0:
            scp[j - _NBUF].wait()
        gcp[j] = gather(j)
        if j - _LAG >= 0:
            gcp[j - _LAG].wait()
            scp[j - _LAG] = scatter(j - _LAG)
    for j in range(_NCHUNK - _LAG, _NCHUNK):
        gcp[j].wait()
        scp[j] = scatter(j)

    boff = b * N
    for t in range(_RPW // _L):
        ord_v[pl.ds(t * _L, _L)] = idx_v[pl.ds(t * _L, _L)] - boff
    ocp = pltpu.async_copy(ord_v, out_o.at[b, pl.ds(r0, _RPW)], osem)

    tcopies = [
        pltpu.async_copy(tf_hbm.at[idx_v.at[pl.ds(j * _CHUNK, _CHUNK)]],
                         tout_v.at[pl.ds(j * _CHUNK, _CHUNK)], tsem)
        for j in range(_NCHUNK)
    ]

    for j in range(max(0, _NCHUNK - _NBUF), _NCHUNK):
        scp[j].wait()
    for cp in tcopies:
        cp.wait()
    pltpu.sync_copy(tout_v, out_t.at[b, pl.ds(r0, _RPW)])
    ocp.wait()


def kernel(patches, tokens):
    pf = patches.reshape(B * N, D)
    tf = tokens.reshape(B * N)
    idx = jnp.asarray(_FLAT_IDX)
    out_p, out_t, out_o = _shuffle_sc(pf, tf, idx)
    return (out_p, out_t, out_o)
